# baseline (speedup 1.0000x reference)
import functools

import jax
import jax.numpy as jnp
from jax import lax
from jax.experimental import pallas as pl
from jax.experimental.pallas import tpu as pltpu
from jax.experimental.pallas import tpu_sc as plsc

BATCH = 16384
D = 8
SR = 128
NROW = 1000000
NSUP = NROW * D // SR
NC, NS, L = 2, 16, 16
NW = NC * NS
BPW = BATCH // NW
CHUNK = 128
NCH = BPW // CHUNK
GPC = CHUNK // L

_OFF_W1 = 0
_OFF_B1 = 136
_OFF_W2 = 144
_OFF_B2 = 176
_OFF_W3 = 180
_OFF_B3 = 184
_NPAR = 192


def _ncf_body(uidx_hbm, iidx_hbm, time_hbm, utab_hbm, itab_hbm, par_hbm,
              out_hbm,
              uidx_v, iidx_v, usup_v, isup_v, time_v,
              ub0, ub1, ib0, ib1, par_v, out_v,
              sem_u, sem_i):
    wid = lax.axis_index("s") * NC + lax.axis_index("c")
    base = wid * BPW

    pltpu.sync_copy(uidx_hbm.at[pl.ds(base, BPW)], uidx_v)
    pltpu.sync_copy(iidx_hbm.at[pl.ds(base, BPW)], iidx_v)

    for q in range(BPW // L):
        sl = pl.ds(q * L, L)
        usup_v[q // GPC, pl.ds((q % GPC) * L, L)] = uidx_v[sl] >> 4
        isup_v[q // GPC, pl.ds((q % GPC) * L, L)] = iidx_v[sl] >> 4

    pltpu.sync_copy(time_hbm.at[pl.ds(base, BPW)], time_v)
    pltpu.sync_copy(par_hbm, par_v)

    ubufs = [ub0, ub1]
    ibufs = [ib0, ib1]

    def fire(c):
        cu = pltpu.async_copy(utab_hbm.at[usup_v.at[c]], ubufs[c % 2], sem_u)
        ci = pltpu.async_copy(itab_hbm.at[isup_v.at[c]], ibufs[c % 2], sem_i)
        return cu, ci

    def wv(p):
        return par_v[p // 8, pl.ds((p % 8) * L, L)]

    W1 = [[wv(_OFF_W1 + j * 17 + k) for k in range(17)] for j in range(8)]
    b1 = [wv(_OFF_B1 + j) for j in range(8)]
    W2 = [[wv(_OFF_W2 + j * 8 + k) for k in range(8)] for j in range(4)]
    b2 = [wv(_OFF_B2 + j) for j in range(4)]
    W3 = [wv(_OFF_W3 + k) for k in range(4)]
    b3 = wv(_OFF_B3)

    iota = lax.iota(jnp.int32, L)
    pend = fire(0)

    for c in range(NCH):
        cu, ci = pend
        cu.wait()
        ci.wait()
        if c + 1 < NCH:
            pend = fire(c + 1)
        ub = ubufs[c % 2]
        ib = ibufs[c % 2]

        def group(g, carry, c=c, ub=ub, ib=ib):
            r0 = c * CHUNK + g * L
            rid = g * L + iota
            t = time_v[pl.ds(r0, L)]
            ulow = (uidx_v[pl.ds(r0, L)] & 15) * D
            ilow = (iidx_v[pl.ds(r0, L)] & 15) * D
            xu = [plsc.load_gather(ub, [rid, ulow + d]) for d in range(D)]
            xi = [plsc.load_gather(ib, [rid, ilow + d]) for d in range(D)]
            h1 = []
            for j in range(8):
                acc = t * W1[j][16] + b1[j]
                for k in range(8):
                    acc = acc + xu[k] * W1[j][k]
                for k in range(8):
                    acc = acc + xi[k] * W1[j][8 + k]
                h1.append(jnp.maximum(acc, 0.0))
            h2 = []
            for j in range(4):
                acc = h1[0] * W2[j][0] + b2[j]
                for k in range(1, 8):
                    acc = acc + h1[k] * W2[j][k]
                h2.append(jnp.maximum(acc, 0.0))
            p = h2[0] * W3[0] + b3
            for k in range(1, 4):
                p = p + h2[k] * W3[k]
            out_v[pl.ds(r0, L)] = p
            return carry

        lax.fori_loop(0, GPC, group, 0)

    pltpu.sync_copy(out_v, out_hbm.at[pl.ds(base, BPW)])


@jax.jit
def _ncf(uidx, iidx, time_input, utab, itab, par):
    f = pl.kernel(
        _ncf_body,
        out_type=jax.ShapeDtypeStruct((BATCH,), jnp.float32),
        mesh=plsc.VectorSubcoreMesh(core_axis_name="c", subcore_axis_name="s",
                                    num_cores=NC, num_subcores=NS),
        compiler_params=pltpu.CompilerParams(needs_layout_passes=False),
        scratch_types=[
            pltpu.VMEM((BPW,), jnp.int32),
            pltpu.VMEM((BPW,), jnp.int32),
            pltpu.VMEM((NCH, CHUNK), jnp.int32),
            pltpu.VMEM((NCH, CHUNK), jnp.int32),
            pltpu.VMEM((BPW,), jnp.float32),
            pltpu.VMEM((CHUNK, SR), jnp.float32),
            pltpu.VMEM((CHUNK, SR), jnp.float32),
            pltpu.VMEM((CHUNK, SR), jnp.float32),
            pltpu.VMEM((CHUNK, SR), jnp.float32),
            pltpu.VMEM((_NPAR // 8, SR), jnp.float32),
            pltpu.VMEM((BPW,), jnp.float32),
            pltpu.SemaphoreType.DMA,
            pltpu.SemaphoreType.DMA,
        ],
    )
    return f(uidx, iidx, time_input, utab, itab, par)


def kernel(user_input, item_input, time_input, user_table, item_table,
           W1, b1, W2, b2, W3, b3):
    par = jnp.concatenate([
        W1.reshape(-1), b1, W2.reshape(-1), b2, W3.reshape(-1), b3,
        jnp.zeros((_NPAR - 185,), jnp.float32)])
    par = jnp.tile(par[:, None], (1, L)).reshape(_NPAR // 8, SR)
    utab = user_table.reshape(NSUP, SR)
    itab = item_table.reshape(NSUP, SR)
    pred = _ncf(user_input, item_input, time_input, utab, itab, par)
    return pred.reshape(BATCH, 1)


# speedup vs baseline: 10.3168x; 10.3168x over previous
import jax
import jax.numpy as jnp
from jax import lax
from jax.experimental import pallas as pl
from jax.experimental.pallas import tpu as pltpu
from jax.experimental.pallas import tpu_sc as plsc

BATCH = 16384
D = 8
TW = 128
NC, NS, L = 2, 16, 16
NW = NC * NS
BPW = BATCH // NW
NG = BPW // L

_OFF_W1 = 0
_OFF_B1 = 136
_OFF_W2 = 144
_OFF_B2 = 176
_OFF_W3 = 180
_OFF_B3 = 184
_NPAR = 192


def _ncf_body(uidx_hbm, iidx_hbm, time_hbm, utab_hbm, itab_hbm, par_hbm,
              out_hbm,
              uidx_v, iidx_v, time_v,
              ub0, ub1, ib0, ib1, par_v, out_v,
              sem0, sem1):
    wid = lax.axis_index("s") * NC + lax.axis_index("c")
    base = wid * BPW

    pltpu.sync_copy(uidx_hbm.at[pl.ds(base, BPW)], uidx_v)
    pltpu.sync_copy(iidx_hbm.at[pl.ds(base, BPW)], iidx_v)
    pltpu.sync_copy(time_hbm.at[pl.ds(base, BPW)], time_v)
    pltpu.sync_copy(par_hbm, par_v)

    ubufs = [ub0, ub1]
    ibufs = [ib0, ib1]
    sems = [sem0, sem1]

    def fire(g, par_sel):
        uvec = uidx_v[pl.ds(g * L, L)]
        ivec = iidx_v[pl.ds(g * L, L)]
        sem = sems[par_sel]
        for j in range(L):
            uo = pl.multiple_of((uvec[j] >> 7) * TW, TW)
            io = pl.multiple_of((ivec[j] >> 7) * TW, TW)
            pltpu.async_copy(utab_hbm.at[:, pl.ds(uo, TW)],
                             ubufs[par_sel].at[j], sem)
            pltpu.async_copy(itab_hbm.at[:, pl.ds(io, TW)],
                             ibufs[par_sel].at[j], sem)

    def drain(par_sel):
        sem = sems[par_sel]
        for _ in range(2 * L):
            pltpu.make_async_copy(
                utab_hbm.at[:, pl.ds(0, TW)],
                ubufs[par_sel].at[0], sem).wait()

    def wv(p):
        return par_v[p // 8, pl.ds((p % 8) * L, L)]

    W1 = [[wv(_OFF_W1 + j * 17 + k) for k in range(17)] for j in range(8)]
    b1 = [wv(_OFF_B1 + j) for j in range(8)]
    W2 = [[wv(_OFF_W2 + j * 8 + k) for k in range(8)] for j in range(4)]
    b2 = [wv(_OFF_B2 + j) for j in range(4)]
    W3 = [wv(_OFF_W3 + k) for k in range(4)]
    b3 = wv(_OFF_B3)

    iota = lax.iota(jnp.int32, L)

    fire(0, 0)

    def step(g, par_sel, carry):
        @pl.when(g + 1 < NG)
        def _():
            fire(g + 1, 1 - par_sel)

        drain(par_sel)

        sl = pl.ds(g * L, L)
        t = time_v[sl]
        ulow = uidx_v[sl] & (TW - 1)
        ilow = iidx_v[sl] & (TW - 1)
        ub = ubufs[par_sel]
        ib = ibufs[par_sel]
        xu = [plsc.load_gather(ub, [iota, jnp.full((L,), d, jnp.int32), ulow])
              for d in range(D)]
        xi = [plsc.load_gather(ib, [iota, jnp.full((L,), d, jnp.int32), ilow])
              for d in range(D)]
        h1 = []
        for j in range(8):
            acc = t * W1[j][16] + b1[j]
            for k in range(8):
                acc = acc + xu[k] * W1[j][k]
            for k in range(8):
                acc = acc + xi[k] * W1[j][8 + k]
            h1.append(jnp.maximum(acc, 0.0))
        h2 = []
        for j in range(4):
            acc = h1[0] * W2[j][0] + b2[j]
            for k in range(1, 8):
                acc = acc + h1[k] * W2[j][k]
            h2.append(jnp.maximum(acc, 0.0))
        p = h2[0] * W3[0] + b3
        for k in range(1, 4):
            p = p + h2[k] * W3[k]
        out_v[sl] = p
        return carry

    def step_even(m, carry):
        step(2 * m, 0, carry)
        return step(2 * m + 1, 1, carry)

    lax.fori_loop(0, NG // 2, step_even, 0)

    pltpu.sync_copy(out_v, out_hbm.at[pl.ds(base, BPW)])


@jax.jit
def _ncf(uidx, iidx, time_input, utabT, itabT, par):
    f = pl.kernel(
        _ncf_body,
        out_type=jax.ShapeDtypeStruct((BATCH,), jnp.float32),
        mesh=plsc.VectorSubcoreMesh(core_axis_name="c", subcore_axis_name="s",
                                    num_cores=NC, num_subcores=NS),
        compiler_params=pltpu.CompilerParams(needs_layout_passes=False),
        scratch_types=[
            pltpu.VMEM((BPW,), jnp.int32),
            pltpu.VMEM((BPW,), jnp.int32),
            pltpu.VMEM((BPW,), jnp.float32),
            pltpu.VMEM((L, D, TW), jnp.float32),
            pltpu.VMEM((L, D, TW), jnp.float32),
            pltpu.VMEM((L, D, TW), jnp.float32),
            pltpu.VMEM((L, D, TW), jnp.float32),
            pltpu.VMEM((_NPAR // 8, L * 8), jnp.float32),
            pltpu.VMEM((BPW,), jnp.float32),
            pltpu.SemaphoreType.DMA,
            pltpu.SemaphoreType.DMA,
        ],
    )
    return f(uidx, iidx, time_input, utabT, itabT, par)


def kernel(user_input, item_input, time_input, user_table, item_table,
           W1, b1, W2, b2, W3, b3):
    par = jnp.concatenate([
        W1.reshape(-1), b1, W2.reshape(-1), b2, W3.reshape(-1), b3,
        jnp.zeros((_NPAR - 185,), jnp.float32)])
    par = jnp.tile(par[:, None], (1, L)).reshape(_NPAR // 8, L * 8)
    pred = _ncf(user_input, item_input, time_input,
                user_table.T, item_table.T, par)
    return pred.reshape(BATCH, 1)
